# baseline (speedup 1.0000x reference)
import functools

import jax
import jax.numpy as jnp
from jax import lax
from jax.experimental import pallas as pl
from jax.experimental.pallas import tpu as pltpu
from jax.experimental.pallas import tpu_sc as plsc

B = 16384
NNEG = 20
D = 128
L = 16
NC = 2
NS = 16
NW = NC * NS
EPW = B // NW
CH = 8
NCH = EPW // CH
NBUF = 4
NVREG = D // L


def _log_sigmoid(x):
    t = jnp.exp(-jnp.abs(x))
    z = t / (t + 2.0)
    z2 = z * z
    p = 1.0 + z2 * (0.33333333 + z2 * (0.2 + z2 * (0.14285714 + z2 * 0.11111111)))
    return jnp.minimum(x, 0.0) - 2.0 * z * p


def _body(upos_h, vpos_h, vnegf_h, uw_h, vw_h, out_h,
          upos_v, vpos_v, vneg_v, u_buf, v_buf, neg_buf,
          out_v, sems):
    wid = lax.axis_index("s") * NC + lax.axis_index("c")
    base = wid * EPW

    pltpu.sync_copy(upos_h.at[pl.ds(base, EPW)], upos_v)
    pltpu.sync_copy(vpos_h.at[pl.ds(base, EPW)], vpos_v)
    pltpu.sync_copy(vnegf_h.at[pl.ds(base * NNEG, EPW * NNEG)], vneg_v)

    def copies(c, b):
        sem = sems.at[b]
        nbase = c * CH * NNEG
        return [
            (uw_h.at[upos_v.at[pl.ds(c * CH, CH)]], u_buf.at[b], sem),
            (vw_h.at[vpos_v.at[pl.ds(c * CH, CH)]], v_buf.at[b], sem),
            (vw_h.at[vneg_v.at[pl.ds(nbase, 128)]],
             neg_buf.at[b].at[pl.ds(0, 128)], sem),
            (vw_h.at[vneg_v.at[pl.ds(nbase + 128, 32)]],
             neg_buf.at[b].at[pl.ds(128, 32)], sem),
        ]

    def fire(c, b):
        for src, dst, s in copies(c, b):
            pltpu.async_copy(src, dst, s)

    def wait(c, b):
        for src, dst, s in copies(c, b):
            pltpu.make_async_copy(src, dst, s).wait()

    lanes = lax.iota(jnp.int32, L)
    lane0 = lanes == 0
    low5 = (lanes < NNEG - L + 1).astype(jnp.float32)
    zero = jnp.zeros((L,), jnp.float32)

    def compute(c, b):
        ub, vb, nb = u_buf.at[b], v_buf.at[b], neg_buf.at[b]

        @plsc.parallel_loop(0, CH, unroll=4)
        def elem(e):
            us = [ub[e, pl.ds(j * L, L)] for j in range(NVREG)]

            def dot(ref, row):
                acc = us[0] * ref[row, pl.ds(0, L)]
                for j in range(1, NVREG):
                    acc = acc + us[j] * ref[row, pl.ds(j * L, L)]
                return jnp.sum(acc)

            vec2 = jnp.where(lane0, jnp.full((L,), dot(vb, e), jnp.float32), zero)
            vec1 = zero
            row = e * NNEG
            for n in range(L):
                sn = dot(nb, row + n)
                vec1 = jnp.where(lanes == n, jnp.full((L,), sn, jnp.float32), vec1)
            for n in range(L, NNEG):
                sn = dot(nb, row + n)
                vec2 = jnp.where(lanes == n - L + 1,
                                 jnp.full((L,), sn, jnp.float32), vec2)

            ls1 = _log_sigmoid(-vec1)
            vec2s = jnp.where(lane0, vec2, -vec2)
            ls2 = _log_sigmoid(vec2s) * low5
            tot = jnp.sum(ls1 + ls2)
            plsc.store_scatter(out_v, [jnp.full((L,), c * CH + e, jnp.int32)],
                               jnp.full((L,), -tot, jnp.float32), mask=lane0)

    for b in range(NBUF - 1):
        fire(b, b)

    def ring(i, carry):
        for b in range(NBUF):
            c = i * NBUF + b
            wait(c, b)
            compute(c, b)

            @pl.when(c + NBUF - 1 < NCH)
            def _():
                fire(c + NBUF - 1, (b + NBUF - 1) % NBUF)

        return carry

    lax.fori_loop(0, NCH // NBUF, ring, 0)

    pltpu.sync_copy(out_v, out_h.at[pl.ds(base, EPW)])


@functools.partial(
    pl.kernel,
    out_type=jax.ShapeDtypeStruct((B,), jnp.float32),
    mesh=plsc.VectorSubcoreMesh(core_axis_name="c", subcore_axis_name="s",
                                num_cores=NC, num_subcores=NS),
    compiler_params=pltpu.CompilerParams(needs_layout_passes=False),
    scratch_types=[
        pltpu.VMEM((EPW,), jnp.int32),
        pltpu.VMEM((EPW,), jnp.int32),
        pltpu.VMEM((EPW * NNEG,), jnp.int32),
        pltpu.VMEM((NBUF, CH, D), jnp.float32),
        pltpu.VMEM((NBUF, CH, D), jnp.float32),
        pltpu.VMEM((NBUF, CH * NNEG, D), jnp.float32),
        pltpu.VMEM((EPW,), jnp.float32),
        pltpu.SemaphoreType.DMA((NBUF,)),
    ],
)
def _skipgram(*refs):
    _body(*refs)


def kernel(u_pos, v_pos, v_neg, u_weight, v_weight):
    return _skipgram(u_pos, v_pos, v_neg.reshape(-1), u_weight, v_weight)


# speedup vs baseline: 1.3196x; 1.3196x over previous
import functools

import jax
import jax.numpy as jnp
from jax import lax
from jax.experimental import pallas as pl
from jax.experimental.pallas import tpu as pltpu
from jax.experimental.pallas import tpu_sc as plsc

B = 16384
NNEG = 20
D = 128
L = 16
NC = 2
NS = 16
NW = NC * NS
EPW = B // NW
CH = 8
NCH = EPW // CH
NBUF = 4
NVREG = D // L


def _log_sigmoid(x):
    t = jnp.exp(-jnp.abs(x))
    z = t / (t + 2.0)
    z2 = z * z
    p = 1.0 + z2 * (0.33333333 + z2 * (0.2 + z2 * (0.14285714 + z2 * 0.11111111)))
    return jnp.minimum(x, 0.0) - 2.0 * z * p


def _body(upos_h, vpos_h, vnegf_h, uw_h, vw_h, out_h,
          upos_v, vpos_v, vneg_v, u_buf, v_buf, neg_buf,
          out_v, sems):
    wid = lax.axis_index("s") * NC + lax.axis_index("c")
    base = wid * EPW

    pltpu.sync_copy(upos_h.at[pl.ds(base, EPW)], upos_v)
    pltpu.sync_copy(vpos_h.at[pl.ds(base, EPW)], vpos_v)
    pltpu.sync_copy(vnegf_h.at[pl.ds(base * NNEG, EPW * NNEG)], vneg_v)

    def copies(c, b):
        sem = sems.at[b]
        nbase = c * CH * NNEG
        return [
            (uw_h.at[upos_v.at[pl.ds(c * CH, CH)]], u_buf.at[b], sem),
            (vw_h.at[vpos_v.at[pl.ds(c * CH, CH)]], v_buf.at[b], sem),
            (vw_h.at[vneg_v.at[pl.ds(nbase, 128)]],
             neg_buf.at[b].at[pl.ds(0, 128)], sem),
            (vw_h.at[vneg_v.at[pl.ds(nbase + 128, 32)]],
             neg_buf.at[b].at[pl.ds(128, 32)], sem),
        ]

    def fire(c, b):
        for src, dst, s in copies(c, b):
            pltpu.async_copy(src, dst, s)

    def wait(c, b):
        for src, dst, s in copies(c, b):
            pltpu.make_async_copy(src, dst, s).wait()

    lanes = lax.iota(jnp.int32, L)
    lane0 = lanes == 0
    low5 = (lanes < NNEG - L + 1).astype(jnp.float32)
    zero = jnp.zeros((L,), jnp.float32)

    def compute(c, b):
        ub, vb, nb = u_buf.at[b], v_buf.at[b], neg_buf.at[b]

        @plsc.parallel_loop(0, CH, unroll=2)
        def elem(e):
            us = [ub[e, pl.ds(j * L, L)] for j in range(NVREG)]

            def dot(ref, row):
                acc = us[0] * ref[row, pl.ds(0, L)]
                for j in range(1, NVREG):
                    acc = acc + us[j] * ref[row, pl.ds(j * L, L)]
                return jnp.sum(acc)

            vec2 = jnp.where(lane0, jnp.full((L,), dot(vb, e), jnp.float32), zero)
            vec1 = zero
            row = e * NNEG
            for n in range(L):
                sn = dot(nb, row + n)
                vec1 = jnp.where(lanes == n, jnp.full((L,), sn, jnp.float32), vec1)
            for n in range(L, NNEG):
                sn = dot(nb, row + n)
                vec2 = jnp.where(lanes == n - L + 1,
                                 jnp.full((L,), sn, jnp.float32), vec2)

            ls1 = _log_sigmoid(-vec1)
            vec2s = jnp.where(lane0, vec2, -vec2)
            ls2 = _log_sigmoid(vec2s) * low5
            tot = jnp.sum(ls1 + ls2)
            plsc.store_scatter(out_v, [jnp.full((L,), c * CH + e, jnp.int32)],
                               jnp.full((L,), -tot, jnp.float32), mask=lane0)

    for b in range(NBUF - 1):
        fire(b, b)

    def ring(i, carry):
        for b in range(NBUF):
            c = i * NBUF + b
            wait(c, b)
            compute(c, b)

            @pl.when(c + NBUF - 1 < NCH)
            def _():
                fire(c + NBUF - 1, (b + NBUF - 1) % NBUF)

        return carry

    lax.fori_loop(0, NCH // NBUF, ring, 0)

    pltpu.sync_copy(out_v, out_h.at[pl.ds(base, EPW)])


@functools.partial(
    pl.kernel,
    out_type=jax.ShapeDtypeStruct((B,), jnp.float32),
    mesh=plsc.VectorSubcoreMesh(core_axis_name="c", subcore_axis_name="s",
                                num_cores=NC, num_subcores=NS),
    compiler_params=pltpu.CompilerParams(needs_layout_passes=False),
    scratch_types=[
        pltpu.VMEM((EPW,), jnp.int32),
        pltpu.VMEM((EPW,), jnp.int32),
        pltpu.VMEM((EPW * NNEG,), jnp.int32),
        pltpu.VMEM((NBUF, CH, D), jnp.float32),
        pltpu.VMEM((NBUF, CH, D), jnp.float32),
        pltpu.VMEM((NBUF, CH * NNEG, D), jnp.float32),
        pltpu.VMEM((EPW,), jnp.float32),
        pltpu.SemaphoreType.DMA((NBUF,)),
    ],
)
def _skipgram(*refs):
    _body(*refs)


def kernel(u_pos, v_pos, v_neg, u_weight, v_weight):
    return _skipgram(u_pos, v_pos, v_neg.reshape(-1), u_weight, v_weight)
